# Initial kernel scaffold; baseline (speedup 1.0000x reference)
import jax
import jax.numpy as jnp
from jax.experimental import pallas as pl


def kernel(xyz, sparse_xyz, sparse_flow):
    raise NotImplementedError("write your pallas kernel here")


# speedup vs baseline: 27.5368x; 27.5368x over previous
import jax
import jax.numpy as jnp
from jax import lax
from jax.experimental import pallas as pl

_TILE_N = 512


def _knn_interp_kernel(xyz_t_ref, sxyz_ref, sflow_ref, out_ref):
    q = xyz_t_ref[0]
    s = sxyz_ref[0]
    f = sflow_ref[0]
    T = q.shape[0]
    S = s.shape[1]

    qc = [q[:, c:c + 1] for c in range(3)]
    sc = [s[c:c + 1, :] for c in range(3)]

    qq = qc[0] * qc[0] + qc[1] * qc[1] + qc[2] * qc[2]
    ss = sc[0] * sc[0] + sc[1] * sc[1] + sc[2] * sc[2]
    qb = [x.astype(jnp.bfloat16).astype(jnp.float32) for x in qc]
    sb = [x.astype(jnp.bfloat16).astype(jnp.float32) for x in sc]
    cross = qb[0] * sb[0] + qb[1] * sb[1] + qb[2] * sb[2]
    d_sel = (qq + ss) - 2.0 * cross

    d_ex = jnp.zeros((T, S), jnp.float32)
    for c in range(3):
        diff = qc[c] - sc[c]
        d_ex = d_ex + diff * diff

    iota = lax.broadcasted_iota(jnp.int32, (T, S), 1)
    inf = jnp.float32(jnp.inf)

    recip = []
    masks = []
    for _ in range(3):
        mk = jnp.min(d_sel, axis=1, keepdims=True)
        cand = jnp.where(d_sel == mk, iota, S)
        ik = jnp.min(cand, axis=1, keepdims=True)
        mask = iota == ik
        d2k = jnp.sum(jnp.where(mask, d_ex, 0.0), axis=1, keepdims=True)
        recip.append(1.0 / jnp.maximum(jnp.sqrt(d2k), 1e-10))
        masks.append(mask)
        d_sel = jnp.where(mask, inf, d_sel)

    norm = recip[0] + recip[1] + recip[2]
    w = jnp.where(masks[0], recip[0] / norm, 0.0)
    w = jnp.where(masks[1], recip[1] / norm, w)
    w = jnp.where(masks[2], recip[2] / norm, w)

    for c in range(3):
        out_ref[0, :, c] = jnp.sum(w * f[c:c + 1, :], axis=1)


def kernel(xyz, sparse_xyz, sparse_flow):
    B, C, N = xyz.shape
    _, _, S = sparse_xyz.shape
    xyz_t = jnp.transpose(xyz, (0, 2, 1))

    grid = (B, N // _TILE_N)
    out = pl.pallas_call(
        _knn_interp_kernel,
        grid=grid,
        in_specs=[
            pl.BlockSpec((1, _TILE_N, C), lambda b, i: (b, i, 0)),
            pl.BlockSpec((1, C, S), lambda b, i: (b, 0, 0)),
            pl.BlockSpec((1, C, S), lambda b, i: (b, 0, 0)),
        ],
        out_specs=pl.BlockSpec((1, _TILE_N, C), lambda b, i: (b, i, 0)),
        out_shape=jax.ShapeDtypeStruct((B, N, C), jnp.float32),
    )(xyz_t, sparse_xyz, sparse_flow)
    return jnp.transpose(out, (0, 2, 1))


# speedup vs baseline: 32.8396x; 1.1926x over previous
import jax
import jax.numpy as jnp
from jax import lax
from jax.experimental import pallas as pl

_TILE_N = 512


def _knn_interp_kernel(xyz_t_ref, qneg2_ref, sxyz_ref, sneg_ref, sflow_ref,
                       out_ref):
    q = xyz_t_ref[0]
    s = sxyz_ref[0]
    f = sflow_ref[0]
    T = q.shape[0]
    S = s.shape[1]

    qc = [q[:, c:c + 1] for c in range(3)]
    sc = [s[c:c + 1, :] for c in range(3)]

    qq = qc[0] * qc[0] + qc[1] * qc[1] + qc[2] * qc[2]
    ss = sc[0] * sc[0] + sc[1] * sc[1] + sc[2] * sc[2]
    crossm = lax.dot_general(
        qneg2_ref[0], sneg_ref[0], (((1,), (0,)), ((), ())),
        preferred_element_type=jnp.float32)
    d_sel = (qq + ss) + crossm

    d_ex = jnp.zeros((T, S), jnp.float32)
    for c in range(3):
        diff = qc[c] - sc[c]
        d_ex = d_ex + diff * diff

    iota_f = lax.broadcasted_iota(jnp.int32, (T, S), 1).astype(jnp.float32)
    inf = jnp.float32(jnp.inf)
    big = jnp.float32(1e9)

    recip = []
    masks = []
    for _ in range(3):
        mk = jnp.min(d_sel, axis=1, keepdims=True)
        cand = jnp.where(d_sel == mk, iota_f, big)
        ik = jnp.min(cand, axis=1, keepdims=True)
        mask = cand == ik
        d2k = jnp.sum(jnp.where(mask, d_ex, 0.0), axis=1, keepdims=True)
        recip.append(1.0 / jnp.maximum(jnp.sqrt(d2k), 1e-10))
        masks.append(mask)
        d_sel = jnp.where(mask, inf, d_sel)

    norm = recip[0] + recip[1] + recip[2]
    w = jnp.where(masks[0], recip[0] / norm, 0.0)
    w = jnp.where(masks[1], recip[1] / norm, w)
    w = jnp.where(masks[2], recip[2] / norm, w)

    for c in range(3):
        out_ref[0, :, c] = jnp.sum(w * f[c:c + 1, :], axis=1)


def kernel(xyz, sparse_xyz, sparse_flow):
    B, C, N = xyz.shape
    _, _, S = sparse_xyz.shape
    xyz_t = jnp.transpose(xyz, (0, 2, 1))
    qb = xyz_t.astype(jnp.bfloat16) * jnp.bfloat16(-2.0)
    qneg2 = jnp.pad(qb, ((0, 0), (0, 0), (0, 5)))
    sneg = jnp.pad(sparse_xyz.astype(jnp.bfloat16),
                   ((0, 0), (0, 5), (0, 0)))

    grid = (B, N // _TILE_N)
    out = pl.pallas_call(
        _knn_interp_kernel,
        grid=grid,
        in_specs=[
            pl.BlockSpec((1, _TILE_N, C), lambda b, i: (b, i, 0)),
            pl.BlockSpec((1, _TILE_N, 8), lambda b, i: (b, i, 0)),
            pl.BlockSpec((1, C, S), lambda b, i: (b, 0, 0)),
            pl.BlockSpec((1, 8, S), lambda b, i: (b, 0, 0)),
            pl.BlockSpec((1, C, S), lambda b, i: (b, 0, 0)),
        ],
        out_specs=pl.BlockSpec((1, _TILE_N, C), lambda b, i: (b, i, 0)),
        out_shape=jax.ShapeDtypeStruct((B, N, C), jnp.float32),
    )(xyz_t, qneg2, sparse_xyz, sneg, sparse_flow)
    return jnp.transpose(out, (0, 2, 1))
